# baseline (speedup 1.0000x reference)
import functools

import jax
import jax.numpy as jnp
from jax.experimental import pallas as pl
from jax.experimental.pallas import tpu as pltpu


def _ceil_div(a, b):
    return (a + b - 1) // b


def _ones_tile(rows):
    return (jax.lax.broadcasted_iota(
        jnp.int32, (rows, 128), 1) == 0).astype(jnp.float32)


def _proj_kernel(x_ref, w_ref, asrc_ref, adst_ref, wh_ref, es_ref, ed_ref,
                 *, H, D):
    x = x_ref[...]
    ones = _ones_tile(x.shape[0])
    for h in range(H):
        whb = jnp.dot(x, w_ref[h], preferred_element_type=jnp.float32)
        wh_ref[h, :, :D] = whb
        wh_ref[h, :, D:] = ones
        es_ref[h] = jnp.sum(whb * asrc_ref[h][None, :], axis=1)
        ed_ref[h] = jnp.sum(whb * adst_ref[h][None, :], axis=1)


def _proj(x, W, a_src, a_dst, block_rows=1024):
    N, D_in = x.shape
    H, _, D = W.shape
    nr = _ceil_div(N, block_rows)
    out_types = (
        jax.ShapeDtypeStruct((H, N, D + 128), jnp.float32),
        jax.ShapeDtypeStruct((H, N), jnp.float32),
        jax.ShapeDtypeStruct((H, N), jnp.float32),
    )
    return pl.pallas_call(
        functools.partial(_proj_kernel, H=H, D=D),
        grid=(nr,),
        in_specs=[
            pl.BlockSpec((block_rows, D_in), lambda r: (r, 0)),
            pl.BlockSpec((H, D_in, D), lambda r: (0, 0, 0)),
            pl.BlockSpec((H, D), lambda r: (0, 0)),
            pl.BlockSpec((H, D), lambda r: (0, 0)),
        ],
        out_specs=(
            pl.BlockSpec((H, block_rows, D + 128), lambda r: (0, r, 0)),
            pl.BlockSpec((H, block_rows), lambda r: (0, r)),
            pl.BlockSpec((H, block_rows), lambda r: (0, r)),
        ),
        out_shape=out_types,
    )(x, W, a_src, a_dst)


def _rowmax_kernel(adjt_ref, ed_ref, m_ref, adjc_ref, ms_ref,
                   *, H, BC, N, NC, compact):
    c = pl.program_id(1)

    @pl.when(c == 0)
    def _init():
        ms_ref[...] = jnp.full(ms_ref.shape, -1e30, jnp.float32)

    adjt = adjt_ref[...]
    if compact:
        r = pl.program_id(0)
        ilane = r * adjt.shape[1] + jax.lax.broadcasted_iota(
            jnp.int32, (1, adjt.shape[1]), 1)
        adjc_ref[...] = jnp.where(ilane < N, adjt, 0.0).astype(jnp.int8)
    else:
        adjt = adjt.astype(jnp.float32)
    jrow = c * BC + jax.lax.broadcasted_iota(jnp.int32, (BC, 1), 0)
    vmask = (adjt > 0) & (jrow < N)
    for h in range(H):
        t = jnp.where(vmask, ed_ref[h][:, None], -1e30)
        ms_ref[h] = jnp.maximum(ms_ref[h], jnp.max(t, axis=0))

    @pl.when(c == NC - 1)
    def _fin():
        m_ref[...] = ms_ref[...]


def _rowmax(adj, ed, *, compact, npad=None, BR=2048, BC=2048):
    H, N = ed.shape
    NR = _ceil_div(N, BR)
    NC = _ceil_div(N, BC)
    out_shape = [jax.ShapeDtypeStruct((H, N), jnp.float32)]
    out_specs = [pl.BlockSpec((H, BR), lambda r, c: (0, r))]
    if compact:
        out_shape.append(jax.ShapeDtypeStruct((N, npad), jnp.int8))
        out_specs.append(pl.BlockSpec((BC, BR), lambda r, c: (c, r)))
    else:
        out_shape.append(jax.ShapeDtypeStruct((8, 128), jnp.int8))
        out_specs.append(pl.BlockSpec((8, 128), lambda r, c: (0, 0)))
    res = pl.pallas_call(
        functools.partial(_rowmax_kernel, H=H, BC=BC, N=N, NC=NC,
                          compact=compact),
        grid=(NR, NC),
        in_specs=[
            pl.BlockSpec((BC, BR), lambda r, c: (c, r)),
            pl.BlockSpec((H, BC), lambda r, c: (0, c)),
        ],
        out_specs=tuple(out_specs),
        out_shape=tuple(out_shape),
        scratch_shapes=[pltpu.VMEM((H, BR), jnp.float32)],
        compiler_params=pltpu.CompilerParams(
            dimension_semantics=("arbitrary", "arbitrary")),
    )(adj, ed)
    return res


def _agg_kernel(adj_ref, wh_ref, es_ref, ed_ref, mmax_ref, w2_ref, a2s_ref,
                a2d_ref, o_ref, wh2_ref, es2_ref, ed2_ref, acc_ref,
                *, H, D, BR, BC, N, NC, fuse_proj2, concat, elu_out, D2):
    c = pl.program_id(0)
    r = pl.program_id(1)
    rows = pl.ds(r * BR, BR)
    DA = D + 128

    @pl.when(c == 0)
    def _init():
        for h in range(H):
            acc_ref[h, rows, :] = jnp.zeros((BR, DA), jnp.float32)

    adjb = adj_ref[...].astype(jnp.bfloat16)
    col = c * BC + jax.lax.broadcasted_iota(jnp.int32, (1, BC), 1)
    colv = (c * BC + jax.lax.broadcasted_iota(jnp.int32, (BC, 1), 0) < N)

    for h in range(H):
        es = es_ref[h]
        edm = jnp.where(col < N, ed_ref[h][None, :], 0.0)
        mrow = es + mmax_ref[h]
        mrow = jnp.maximum(mrow, 0.2 * mrow)
        esm = es - mrow
        esb = 0.2 * es - mrow
        t = esm[:, None] + edm
        u = esb[:, None] + 0.2 * edm
        p = jnp.exp(jnp.maximum(t, u)).astype(jnp.bfloat16) * adjb
        whc = jnp.where(colv, wh_ref[h], 0.0).astype(jnp.bfloat16)
        acc_ref[h, rows, :] = acc_ref[h, rows, :] + jnp.dot(
            p, whc, preferred_element_type=jnp.float32)

    @pl.when(c == NC - 1)
    def _fin():
        if concat:
            outs = []
            for h in range(H):
                out = acc_ref[h, rows, :D] / acc_ref[h, rows, D][:, None]
                if elu_out:
                    out = jnp.where(out > 0, out,
                                    jnp.exp(jnp.minimum(out, 0.0)) - 1.0)
                if not fuse_proj2:
                    o_ref[:, h * D:(h + 1) * D] = out
                outs.append(out)
        else:
            tot = acc_ref[0, rows, :D] / acc_ref[0, rows, D][:, None]
            for h in range(1, H):
                tot = tot + acc_ref[h, rows, :D] / acc_ref[h, rows, D][:, None]
            out = tot * (1.0 / H)
            if elu_out:
                out = jnp.where(out > 0, out,
                                jnp.exp(jnp.minimum(out, 0.0)) - 1.0)
            o_ref[...] = out
        if fuse_proj2:
            hcat = jnp.concatenate(outs, axis=1)
            wh2 = jnp.dot(hcat, w2_ref[0], preferred_element_type=jnp.float32)
            wh2_ref[0, :, :D2] = wh2
            wh2_ref[0, :, D2:] = _ones_tile(BR)
            es2_ref[0] = jnp.sum(wh2 * a2s_ref[0][None, :], axis=1)
            ed2_ref[0] = jnp.sum(wh2 * a2d_ref[0][None, :], axis=1)


def _gat_agg(adj, wh, es, ed, mmax, W2, a2s, a2d, *, fuse_proj2, concat,
             elu_out, BR=1024, BC=2048):
    H, N, DA = wh.shape
    D = DA - 128
    NR = _ceil_div(N, BR)
    NC = adj.shape[1] // BC
    d_out = H * D if concat else D
    D2 = W2.shape[2]
    if fuse_proj2:
        out_shape = [jax.ShapeDtypeStruct((8, d_out), jnp.float32)]
        out_specs = [pl.BlockSpec((8, d_out), lambda c, r: (0, 0))]
    else:
        out_shape = [jax.ShapeDtypeStruct((N, d_out), jnp.float32)]
        out_specs = [pl.BlockSpec((BR, d_out), lambda c, r: (r, 0))]
    if fuse_proj2:
        out_shape += [
            jax.ShapeDtypeStruct((1, N, D2 + 128), jnp.float32),
            jax.ShapeDtypeStruct((1, N), jnp.float32),
            jax.ShapeDtypeStruct((1, N), jnp.float32),
        ]
        out_specs += [
            pl.BlockSpec((1, BR, D2 + 128), lambda c, r: (0, r, 0)),
            pl.BlockSpec((1, BR), lambda c, r: (0, r)),
            pl.BlockSpec((1, BR), lambda c, r: (0, r)),
        ]
    else:
        out_shape += [
            jax.ShapeDtypeStruct((1, 8, 128), jnp.float32),
            jax.ShapeDtypeStruct((1, 8), jnp.float32),
            jax.ShapeDtypeStruct((1, 8), jnp.float32),
        ]
        out_specs += [
            pl.BlockSpec((1, 8, 128), lambda c, r: (0, 0, 0)),
            pl.BlockSpec((1, 8), lambda c, r: (0, 0)),
            pl.BlockSpec((1, 8), lambda c, r: (0, 0)),
        ]
    return pl.pallas_call(
        functools.partial(_agg_kernel, H=H, D=D, BR=BR, BC=BC, N=N, NC=NC,
                          fuse_proj2=fuse_proj2, concat=concat,
                          elu_out=elu_out, D2=D2),
        grid=(NC, NR),
        in_specs=[
            pl.BlockSpec((BR, BC), lambda c, r: (r, c)),
            pl.BlockSpec((H, BC, DA), lambda c, r: (0, c, 0)),
            pl.BlockSpec((H, BR), lambda c, r: (0, r)),
            pl.BlockSpec((H, BC), lambda c, r: (0, c)),
            pl.BlockSpec((H, BR), lambda c, r: (0, r)),
            pl.BlockSpec(W2.shape, lambda c, r: (0, 0, 0)),
            pl.BlockSpec(a2s.shape, lambda c, r: (0, 0)),
            pl.BlockSpec(a2d.shape, lambda c, r: (0, 0)),
        ],
        out_specs=tuple(out_specs),
        out_shape=tuple(out_shape),
        scratch_shapes=[pltpu.VMEM((H, NR * BR, DA), jnp.float32)],
        compiler_params=pltpu.CompilerParams(
            dimension_semantics=("arbitrary", "arbitrary")),
    )(adj, wh, es, ed, mmax, W2, a2s, a2d)


def kernel(adjacency, feature, W1, a1_src, a1_dst, W2, a2_src, a2_dst):
    npad = _ceil_div(adjacency.shape[0], 2048) * 2048
    wh1, es1, ed1 = _proj(feature, W1, a1_src, a1_dst)
    m1, adjc = _rowmax(adjacency, ed1, compact=True, npad=npad)
    _, wh2, es2, ed2 = _gat_agg(
        adjc, wh1, es1, ed1, m1, W2, a2_src, a2_dst,
        fuse_proj2=True, concat=True, elu_out=True)
    m2, _ = _rowmax(adjc, ed2, compact=False)
    out, _, _, _ = _gat_agg(
        adjc, wh2, es2, ed2, m2, W2, a2_src, a2_dst,
        fuse_proj2=False, concat=False, elu_out=False)
    return out


# speedup vs baseline: 1.0130x; 1.0130x over previous
import functools

import jax
import jax.numpy as jnp
from jax.experimental import pallas as pl
from jax.experimental.pallas import tpu as pltpu


def _ceil_div(a, b):
    return (a + b - 1) // b


def _ones_tile(rows):
    return (jax.lax.broadcasted_iota(
        jnp.int32, (rows, 128), 1) == 0).astype(jnp.float32)


def _proj_kernel(x_ref, w_ref, asrc_ref, adst_ref, wh_ref, es_ref, ed_ref,
                 *, H, D):
    x = x_ref[...]
    ones = _ones_tile(x.shape[0])
    for h in range(H):
        whb = jnp.dot(x, w_ref[h], preferred_element_type=jnp.float32)
        wh_ref[h, :, :D] = whb
        wh_ref[h, :, D:] = ones
        es_ref[h] = jnp.sum(whb * asrc_ref[h][None, :], axis=1)
        ed_ref[h] = jnp.sum(whb * adst_ref[h][None, :], axis=1)


def _proj(x, W, a_src, a_dst, block_rows=1024):
    N, D_in = x.shape
    H, _, D = W.shape
    nr = _ceil_div(N, block_rows)
    out_types = (
        jax.ShapeDtypeStruct((H, N, D + 128), jnp.float32),
        jax.ShapeDtypeStruct((H, N), jnp.float32),
        jax.ShapeDtypeStruct((H, N), jnp.float32),
    )
    return pl.pallas_call(
        functools.partial(_proj_kernel, H=H, D=D),
        grid=(nr,),
        in_specs=[
            pl.BlockSpec((block_rows, D_in), lambda r: (r, 0)),
            pl.BlockSpec((H, D_in, D), lambda r: (0, 0, 0)),
            pl.BlockSpec((H, D), lambda r: (0, 0)),
            pl.BlockSpec((H, D), lambda r: (0, 0)),
        ],
        out_specs=(
            pl.BlockSpec((H, block_rows, D + 128), lambda r: (0, r, 0)),
            pl.BlockSpec((H, block_rows), lambda r: (0, r)),
            pl.BlockSpec((H, block_rows), lambda r: (0, r)),
        ),
        out_shape=out_types,
    )(x, W, a_src, a_dst)


def _rowmax_kernel(adjt_ref, ed_ref, m_ref, adjc_ref, ms_ref,
                   *, H, BC, N, NC, compact):
    c = pl.program_id(1)

    @pl.when(c == 0)
    def _init():
        ms_ref[...] = jnp.full(ms_ref.shape, -1e30, jnp.float32)

    adjt = adjt_ref[...]
    if compact:
        r = pl.program_id(0)
        ilane = r * adjt.shape[1] + jax.lax.broadcasted_iota(
            jnp.int32, (1, adjt.shape[1]), 1)
        adjc_ref[...] = jnp.where(ilane < N, adjt, 0.0).astype(jnp.int8)
    else:
        adjt = adjt.astype(jnp.float32)
    jrow = c * BC + jax.lax.broadcasted_iota(jnp.int32, (BC, 1), 0)
    vmask = (adjt > 0) & (jrow < N)
    for h in range(H):
        t = jnp.where(vmask, ed_ref[h][:, None], -1e30)
        ms_ref[h] = jnp.maximum(ms_ref[h], jnp.max(t, axis=0))

    @pl.when(c == NC - 1)
    def _fin():
        m_ref[...] = ms_ref[...]


def _rowmax(adj, ed, *, compact, npad=None, BR=2048, BC=1024):
    H, N = ed.shape
    NR = _ceil_div(N, BR)
    NC = _ceil_div(N, BC)
    out_shape = [jax.ShapeDtypeStruct((H, N), jnp.float32)]
    out_specs = [pl.BlockSpec((H, BR), lambda r, c: (0, r))]
    if compact:
        out_shape.append(jax.ShapeDtypeStruct((N, npad), jnp.int8))
        out_specs.append(pl.BlockSpec((BC, BR), lambda r, c: (c, r)))
    else:
        out_shape.append(jax.ShapeDtypeStruct((8, 128), jnp.int8))
        out_specs.append(pl.BlockSpec((8, 128), lambda r, c: (0, 0)))
    res = pl.pallas_call(
        functools.partial(_rowmax_kernel, H=H, BC=BC, N=N, NC=NC,
                          compact=compact),
        grid=(NR, NC),
        in_specs=[
            pl.BlockSpec((BC, BR), lambda r, c: (c, r)),
            pl.BlockSpec((H, BC), lambda r, c: (0, c)),
        ],
        out_specs=tuple(out_specs),
        out_shape=tuple(out_shape),
        scratch_shapes=[pltpu.VMEM((H, BR), jnp.float32)],
        compiler_params=pltpu.CompilerParams(
            dimension_semantics=("arbitrary", "arbitrary")),
    )(adj, ed)
    return res


def _agg_kernel(adj_ref, wh_ref, es_ref, ed_ref, mmax_ref, w2_ref, a2s_ref,
                a2d_ref, o_ref, wh2_ref, es2_ref, ed2_ref, acc_ref,
                *, H, D, BR, BC, N, NC, fuse_proj2, concat, elu_out, D2):
    c = pl.program_id(0)
    r = pl.program_id(1)
    rows = pl.ds(r * BR, BR)
    DA = D + 128

    @pl.when(c == 0)
    def _init():
        for h in range(H):
            acc_ref[h, rows, :] = jnp.zeros((BR, DA), jnp.float32)

    adjb = adj_ref[...].astype(jnp.bfloat16)
    col = c * BC + jax.lax.broadcasted_iota(jnp.int32, (1, BC), 1)
    colv = (c * BC + jax.lax.broadcasted_iota(jnp.int32, (BC, 1), 0) < N)

    for h in range(H):
        es = es_ref[h]
        edm = jnp.where(col < N, ed_ref[h][None, :], 0.0)
        mrow = es + mmax_ref[h]
        mrow = jnp.maximum(mrow, 0.2 * mrow)
        esm = es - mrow
        esb = 0.2 * es - mrow
        t = esm[:, None] + edm
        u = esb[:, None] + 0.2 * edm
        p = jnp.exp(jnp.maximum(t, u)).astype(jnp.bfloat16) * adjb
        whc = jnp.where(colv, wh_ref[h], 0.0).astype(jnp.bfloat16)
        acc_ref[h, rows, :] = acc_ref[h, rows, :] + jnp.dot(
            p, whc, preferred_element_type=jnp.float32)

    @pl.when(c == NC - 1)
    def _fin():
        if concat:
            outs = []
            for h in range(H):
                out = acc_ref[h, rows, :D] / acc_ref[h, rows, D][:, None]
                if elu_out:
                    out = jnp.where(out > 0, out,
                                    jnp.exp(jnp.minimum(out, 0.0)) - 1.0)
                if not fuse_proj2:
                    o_ref[:, h * D:(h + 1) * D] = out
                outs.append(out)
        else:
            tot = acc_ref[0, rows, :D] / acc_ref[0, rows, D][:, None]
            for h in range(1, H):
                tot = tot + acc_ref[h, rows, :D] / acc_ref[h, rows, D][:, None]
            out = tot * (1.0 / H)
            if elu_out:
                out = jnp.where(out > 0, out,
                                jnp.exp(jnp.minimum(out, 0.0)) - 1.0)
            o_ref[...] = out
        if fuse_proj2:
            hcat = jnp.concatenate(outs, axis=1)
            wh2 = jnp.dot(hcat, w2_ref[0], preferred_element_type=jnp.float32)
            wh2_ref[0, :, :D2] = wh2
            wh2_ref[0, :, D2:] = _ones_tile(BR)
            es2_ref[0] = jnp.sum(wh2 * a2s_ref[0][None, :], axis=1)
            ed2_ref[0] = jnp.sum(wh2 * a2d_ref[0][None, :], axis=1)


def _gat_agg(adj, wh, es, ed, mmax, W2, a2s, a2d, *, fuse_proj2, concat,
             elu_out, BR=1024, BC=2048):
    H, N, DA = wh.shape
    D = DA - 128
    NR = _ceil_div(N, BR)
    NC = adj.shape[1] // BC
    d_out = H * D if concat else D
    D2 = W2.shape[2]
    if fuse_proj2:
        out_shape = [jax.ShapeDtypeStruct((8, d_out), jnp.float32)]
        out_specs = [pl.BlockSpec((8, d_out), lambda c, r: (0, 0))]
    else:
        out_shape = [jax.ShapeDtypeStruct((N, d_out), jnp.float32)]
        out_specs = [pl.BlockSpec((BR, d_out), lambda c, r: (r, 0))]
    if fuse_proj2:
        out_shape += [
            jax.ShapeDtypeStruct((1, N, D2 + 128), jnp.float32),
            jax.ShapeDtypeStruct((1, N), jnp.float32),
            jax.ShapeDtypeStruct((1, N), jnp.float32),
        ]
        out_specs += [
            pl.BlockSpec((1, BR, D2 + 128), lambda c, r: (0, r, 0)),
            pl.BlockSpec((1, BR), lambda c, r: (0, r)),
            pl.BlockSpec((1, BR), lambda c, r: (0, r)),
        ]
    else:
        out_shape += [
            jax.ShapeDtypeStruct((1, 8, 128), jnp.float32),
            jax.ShapeDtypeStruct((1, 8), jnp.float32),
            jax.ShapeDtypeStruct((1, 8), jnp.float32),
        ]
        out_specs += [
            pl.BlockSpec((1, 8, 128), lambda c, r: (0, 0, 0)),
            pl.BlockSpec((1, 8), lambda c, r: (0, 0)),
            pl.BlockSpec((1, 8), lambda c, r: (0, 0)),
        ]
    return pl.pallas_call(
        functools.partial(_agg_kernel, H=H, D=D, BR=BR, BC=BC, N=N, NC=NC,
                          fuse_proj2=fuse_proj2, concat=concat,
                          elu_out=elu_out, D2=D2),
        grid=(NC, NR),
        in_specs=[
            pl.BlockSpec((BR, BC), lambda c, r: (r, c)),
            pl.BlockSpec((H, BC, DA), lambda c, r: (0, c, 0)),
            pl.BlockSpec((H, BR), lambda c, r: (0, r)),
            pl.BlockSpec((H, BC), lambda c, r: (0, c)),
            pl.BlockSpec((H, BR), lambda c, r: (0, r)),
            pl.BlockSpec(W2.shape, lambda c, r: (0, 0, 0)),
            pl.BlockSpec(a2s.shape, lambda c, r: (0, 0)),
            pl.BlockSpec(a2d.shape, lambda c, r: (0, 0)),
        ],
        out_specs=tuple(out_specs),
        out_shape=tuple(out_shape),
        scratch_shapes=[pltpu.VMEM((H, NR * BR, DA), jnp.float32)],
        compiler_params=pltpu.CompilerParams(
            dimension_semantics=("arbitrary", "arbitrary")),
    )(adj, wh, es, ed, mmax, W2, a2s, a2d)


def kernel(adjacency, feature, W1, a1_src, a1_dst, W2, a2_src, a2_dst):
    npad = _ceil_div(adjacency.shape[0], 2048) * 2048
    wh1, es1, ed1 = _proj(feature, W1, a1_src, a1_dst)
    m1, adjc = _rowmax(adjacency, ed1, compact=True, npad=npad)
    _, wh2, es2, ed2 = _gat_agg(
        adjc, wh1, es1, ed1, m1, W2, a2_src, a2_dst,
        fuse_proj2=True, concat=True, elu_out=True)
    m2, _ = _rowmax(adjc, ed2, compact=False)
    out, _, _, _ = _gat_agg(
        adjc, wh2, es2, ed2, m2, W2, a2_src, a2_dst,
        fuse_proj2=False, concat=False, elu_out=False)
    return out
